# baseline (speedup 1.0000x reference)
import functools

import jax
import jax.numpy as jnp
from jax import lax
from jax.experimental import pallas as pl
from jax.experimental.pallas import tpu as pltpu
from jax.experimental.pallas import tpu_sc as plsc

_VOCAB = 256
_BATCH = 4096
_HIST = 50
_HPAD = 56
_TB = 128
_MB = 32

_SB = 512
_NC, _NS = 2, 16
_NW = _NC * _NS
_CH = 112
_PW = _SB * _HPAD // _NW
_NCHUNK = _PW // _CH
_NBUF = 2


def _logits_body(table_ref, w_ref, b_ref, out_ref):
    out_ref[...] = lax.dot_general(
        table_ref[...], w_ref[...], (((1,), (1,)), ((), ())),
        preferred_element_type=jnp.float32) + b_ref[...]


def _compute_logits(table, W, b):
    return pl.pallas_call(
        _logits_body,
        out_shape=jax.ShapeDtypeStruct((_VOCAB, _VOCAB), jnp.float32),
    )(table, W, b.reshape(1, _VOCAB))


@functools.partial(
    pl.kernel,
    mesh=plsc.VectorSubcoreMesh(core_axis_name="c", subcore_axis_name="s"),
    out_type=jax.ShapeDtypeStruct((_SB * _HPAD, _VOCAB), jnp.float32),
    scratch_types=[
        pltpu.VMEM((_NCHUNK, _CH), jnp.int32),
        pltpu.VMEM((_CH, _VOCAB), jnp.float32),
        pltpu.VMEM((_CH, _VOCAB), jnp.float32),
        pltpu.SemaphoreType.DMA,
        pltpu.SemaphoreType.DMA,
        pltpu.SemaphoreType.DMA,
        pltpu.SemaphoreType.DMA,
    ],
)
def _sc_gather(x_hbm, logits_hbm, out_hbm, idx_v, rows0, rows1,
               g0, g1, o0, o1):
    wid = lax.axis_index("s") * _NC + lax.axis_index("c")
    pltpu.sync_copy(x_hbm.at[wid], idx_v)
    base0 = wid * _PW
    bufs = ((rows0, g0, o0), (rows1, g1, o1))

    def g_start(j, b):
        rows, g, _ = bufs[b]
        pltpu.async_copy(logits_hbm.at[idx_v.at[j]], rows, g)

    def g_wait(j, b):
        rows, g, _ = bufs[b]
        pltpu.make_async_copy(logits_hbm.at[idx_v.at[j]], rows, g).wait()

    def s_start(j, b):
        rows, _, o = bufs[b]
        pltpu.async_copy(rows, out_hbm.at[pl.ds(base0 + j * _CH, _CH)], o)

    def s_wait(j, b):
        rows, _, o = bufs[b]
        pltpu.make_async_copy(
            rows, out_hbm.at[pl.ds(base0 + j * _CH, _CH)], o).wait()

    for b in range(_NBUF):
        g_start(b, b)

    def body(i, carry):
        j0 = i * _NBUF
        for b in range(_NBUF):
            g_wait(j0 + b, b)
            s_start(j0 + b, b)
        for b in range(_NBUF):
            s_wait(j0 + b, b)
            g_start(j0 + _NBUF + b, b)
        return carry

    lax.fori_loop(0, _NCHUNK // _NBUF - 1, body, 0)

    j0 = _NCHUNK - _NBUF
    for b in range(_NBUF):
        g_wait(j0 + b, b)
        s_start(j0 + b, b)
    for b in range(_NBUF):
        s_wait(j0 + b, b)


def _onehot_body(x_ref, logits_ref, out_ref):
    lg = logits_ref[...]
    m = _TB * _HPAD
    idx = x_ref[0]
    oht = (idx == lax.broadcasted_iota(
        jnp.int32, (_VOCAB, m), 0)).astype(jnp.bfloat16)
    acc = lax.dot_general(oht, lg, (((0,), (0,)), ((), ())),
                          preferred_element_type=jnp.float32)
    for k in range(_TB):
        out_ref[k] = acc[k * _HPAD:k * _HPAD + _HIST, :]


def _tc_decode(x2, logits_bf):
    off = _SB // _TB
    return pl.pallas_call(
        _onehot_body,
        grid=((_BATCH - _SB) // _TB,),
        in_specs=[
            pl.BlockSpec((1, 1, _TB * _HPAD), lambda i: (off + i, 0, 0)),
            pl.BlockSpec((_VOCAB, _VOCAB), lambda i: (0, 0)),
        ],
        out_specs=pl.BlockSpec((_TB, _HIST, _VOCAB),
                               lambda i: (off + i, 0, 0)),
        out_shape=jax.ShapeDtypeStruct((_BATCH, _HIST, _VOCAB), jnp.float32),
    )(x2, logits_bf)


def _merge_body(sc_ref, outin_ref, out_ref):
    del outin_ref
    acc = sc_ref[...]
    for k in range(_MB):
        out_ref[k] = acc[k * _HPAD:k * _HPAD + _HIST, :]


def _tc_merge(sc2d, out1):
    return pl.pallas_call(
        _merge_body,
        grid=(_SB // _MB,),
        in_specs=[
            pl.BlockSpec((_MB * _HPAD, _VOCAB), lambda i: (i, 0)),
            pl.BlockSpec(memory_space=pl.ANY),
        ],
        out_specs=pl.BlockSpec((_MB, _HIST, _VOCAB), lambda i: (i, 0, 0)),
        out_shape=jax.ShapeDtypeStruct((_BATCH, _HIST, _VOCAB), jnp.float32),
        input_output_aliases={1: 0},
    )(sc2d, out1)


def kernel(x, table, W, b):
    logits = _compute_logits(table, W, b)
    xp = jnp.pad(x.astype(jnp.int32), ((0, 0), (0, _HPAD - _HIST)))
    xsc = xp[:_SB].reshape(_NW, _NCHUNK, _CH)
    sc2d = _sc_gather(xsc, logits)
    x2 = xp.reshape(_BATCH // _TB, 1, _TB * _HPAD)
    out1 = _tc_decode(x2, logits.astype(jnp.bfloat16))
    return _tc_merge(sc2d, out1)


# speedup vs baseline: 1.5039x; 1.5039x over previous
import functools

import jax
import jax.numpy as jnp
from jax import lax
from jax.experimental import pallas as pl
from jax.experimental.pallas import tpu as pltpu
from jax.experimental.pallas import tpu_sc as plsc

_VOCAB = 256
_BATCH = 4096
_HIST = 50
_HPAD = 56
_TB = 128
_MB = 32

_SB = 128
_NC, _NS = 2, 16
_NW = _NC * _NS
_CH = 112
_PW = _SB * _HPAD // _NW
_NCHUNK = _PW // _CH
_NBUF = 2


def _logits_body(table_ref, w_ref, b_ref, out_ref):
    out_ref[...] = lax.dot_general(
        table_ref[...], w_ref[...], (((1,), (1,)), ((), ())),
        preferred_element_type=jnp.float32) + b_ref[...]


def _compute_logits(table, W, b):
    return pl.pallas_call(
        _logits_body,
        out_shape=jax.ShapeDtypeStruct((_VOCAB, _VOCAB), jnp.float32),
    )(table, W, b.reshape(1, _VOCAB))


@functools.partial(
    pl.kernel,
    mesh=plsc.VectorSubcoreMesh(core_axis_name="c", subcore_axis_name="s"),
    out_type=jax.ShapeDtypeStruct((_SB * _HPAD, _VOCAB), jnp.float32),
    scratch_types=[
        pltpu.VMEM((_NCHUNK, _CH), jnp.int32),
        pltpu.VMEM((_CH, _VOCAB), jnp.float32),
        pltpu.VMEM((_CH, _VOCAB), jnp.float32),
        pltpu.SemaphoreType.DMA,
        pltpu.SemaphoreType.DMA,
        pltpu.SemaphoreType.DMA,
        pltpu.SemaphoreType.DMA,
    ],
)
def _sc_gather(x_hbm, logits_hbm, out_hbm, idx_v, rows0, rows1,
               g0, g1, o0, o1):
    wid = lax.axis_index("s") * _NC + lax.axis_index("c")
    pltpu.sync_copy(x_hbm.at[wid], idx_v)
    base0 = wid * _PW
    bufs = ((rows0, g0, o0), (rows1, g1, o1))

    def g_start(j, b):
        rows, g, _ = bufs[b]
        pltpu.async_copy(logits_hbm.at[idx_v.at[j]], rows, g)

    def g_wait(j, b):
        rows, g, _ = bufs[b]
        pltpu.make_async_copy(logits_hbm.at[idx_v.at[j]], rows, g).wait()

    def s_start(j, b):
        rows, _, o = bufs[b]
        pltpu.async_copy(rows, out_hbm.at[pl.ds(base0 + j * _CH, _CH)], o)

    def s_wait(j, b):
        rows, _, o = bufs[b]
        pltpu.make_async_copy(
            rows, out_hbm.at[pl.ds(base0 + j * _CH, _CH)], o).wait()

    for b in range(_NBUF):
        g_start(b, b)

    def body(i, carry):
        j0 = i * _NBUF
        for b in range(_NBUF):
            g_wait(j0 + b, b)
            s_start(j0 + b, b)
        for b in range(_NBUF):
            s_wait(j0 + b, b)
            g_start(j0 + _NBUF + b, b)
        return carry

    lax.fori_loop(0, _NCHUNK // _NBUF - 1, body, 0)

    j0 = _NCHUNK - _NBUF
    for b in range(_NBUF):
        g_wait(j0 + b, b)
        s_start(j0 + b, b)
    for b in range(_NBUF):
        s_wait(j0 + b, b)


def _onehot_body(x_ref, logits_ref, out_ref):
    lg = logits_ref[...]
    m = _TB * _HPAD
    idx = x_ref[0]
    oht = (idx == lax.broadcasted_iota(
        jnp.int32, (_VOCAB, m), 0)).astype(jnp.bfloat16)
    acc = lax.dot_general(oht, lg, (((0,), (0,)), ((), ())),
                          preferred_element_type=jnp.float32)
    for k in range(_TB):
        out_ref[k] = acc[k * _HPAD:k * _HPAD + _HIST, :]


def _tc_decode(x2, logits_bf):
    off = _SB // _TB
    return pl.pallas_call(
        _onehot_body,
        grid=((_BATCH - _SB) // _TB,),
        in_specs=[
            pl.BlockSpec((1, 1, _TB * _HPAD), lambda i: (off + i, 0, 0)),
            pl.BlockSpec((_VOCAB, _VOCAB), lambda i: (0, 0)),
        ],
        out_specs=pl.BlockSpec((_TB, _HIST, _VOCAB),
                               lambda i: (off + i, 0, 0)),
        out_shape=jax.ShapeDtypeStruct((_BATCH, _HIST, _VOCAB), jnp.float32),
    )(x2, logits_bf)


def _merge_body(sc_ref, outin_ref, out_ref):
    del outin_ref
    acc = sc_ref[...]
    for k in range(_MB):
        out_ref[k] = acc[k * _HPAD:k * _HPAD + _HIST, :]


def _tc_merge(sc2d, out1):
    return pl.pallas_call(
        _merge_body,
        grid=(_SB // _MB,),
        in_specs=[
            pl.BlockSpec((_MB * _HPAD, _VOCAB), lambda i: (i, 0)),
            pl.BlockSpec(memory_space=pl.ANY),
        ],
        out_specs=pl.BlockSpec((_MB, _HIST, _VOCAB), lambda i: (i, 0, 0)),
        out_shape=jax.ShapeDtypeStruct((_BATCH, _HIST, _VOCAB), jnp.float32),
        input_output_aliases={1: 0},
    )(sc2d, out1)


def kernel(x, table, W, b):
    logits = _compute_logits(table, W, b)
    xp = jnp.pad(x.astype(jnp.int32), ((0, 0), (0, _HPAD - _HIST)))
    xsc = xp[:_SB].reshape(_NW, _NCHUNK, _CH)
    sc2d = _sc_gather(xsc, logits)
    x2 = xp.reshape(_BATCH // _TB, 1, _TB * _HPAD)
    out1 = _tc_decode(x2, logits.astype(jnp.bfloat16))
    return _tc_merge(sc2d, out1)
